# baseline (speedup 1.0000x reference)
import jax
import jax.numpy as jnp
from jax import lax
from jax.experimental import pallas as pl
from jax.experimental.pallas import tpu as pltpu
from jax.experimental.pallas import tpu_sc as plsc

VOCAB = 1000
DIM = 64
BATCH = 4096

NC = 2
NS = 16
NW = NC * NS
LANES = 16
B_PER_W = BATCH // NW
GROUPS = B_PER_W // LANES
CHUNKS = DIM // LANES
HALF = B_PER_W // 2

_GATHER_DNUMS = lax.GatherDimensionNumbers(
    offset_dims=(), collapsed_slice_dims=(0,), start_index_map=(0,))


def _lane_perm(x, idx):
    return lax.gather(
        x, idx[:, None], _GATHER_DNUMS, slice_sizes=(1,),
        mode=lax.GatherScatterMode.PROMISE_IN_BOUNDS)


QUARTER = B_PER_W // 4


def _sc_body(ctx_idx_hbm, tgt_idx_hbm, cxt_hbm, emb_hbm, out_hbm,
             cidx_v, tidx_v, crows_v, trows_v, out_v,
             sem_i, sem_a, sem_b, sem_c2, sem_d):
    wid = lax.axis_index("s") * NC + lax.axis_index("c")
    base = wid * B_PER_W

    ci = pltpu.async_copy(ctx_idx_hbm.at[pl.ds(base, B_PER_W)], cidx_v, sem_i)
    ti = pltpu.async_copy(tgt_idx_hbm.at[pl.ds(base, B_PER_W)], tidx_v, sem_i)
    ci.wait()
    ti.wait()
    cp = []
    for h, sem in ((0, sem_a), (1, sem_b), (2, sem_c2), (3, sem_d)):
        sl = pl.ds(h * QUARTER, QUARTER)
        cp.append(pltpu.async_copy(cxt_hbm.at[cidx_v.at[sl]],
                                   crows_v.at[sl], sem))
        cp.append(pltpu.async_copy(emb_hbm.at[tidx_v.at[sl]],
                                   trows_v.at[sl], sem))

    lane = lax.iota(jnp.int32, LANES)
    bitrev = (((lane & 1) << 3) | ((lane & 2) << 1)
              | ((lane & 4) >> 1) | ((lane & 8) >> 3))

    def combine(a, b, sh):
        keep = (lane & sh) == 0
        return jnp.where(keep, a + _lane_perm(a, lane ^ sh),
                         b + _lane_perm(b, lane ^ sh))

    def do_group(g):
        ps = []
        for j in range(LANES):
            r = g * LANES + j
            p = crows_v[r, pl.ds(0, LANES)] * trows_v[r, pl.ds(0, LANES)]
            for k in range(1, CHUNKS):
                p = p + (crows_v[r, pl.ds(k * LANES, LANES)]
                         * trows_v[r, pl.ds(k * LANES, LANES)])
            ps.append(p)
        for sh in (8, 4, 2, 1):
            ps = [combine(ps[2 * i], ps[2 * i + 1], sh)
                  for i in range(len(ps) // 2)]
        out_v[pl.ds(g * LANES, LANES)] = _lane_perm(ps[0], bitrev)

    def loop_body(g, carry):
        for q in range(4):
            @pl.when(g == 2 * q)
            def _(q=q):
                cp[2 * q].wait()
                cp[2 * q + 1].wait()

        do_group(g)
        return carry

    lax.fori_loop(0, GROUPS, loop_body, 0)

    pltpu.sync_copy(out_v, out_hbm.at[pl.ds(base, B_PER_W)])


@jax.jit
def _sc_call(context_idxs, target_idxs, cxt_table, emb_table):
    mesh = plsc.VectorSubcoreMesh(core_axis_name="c", subcore_axis_name="s")
    kern = pl.kernel(
        _sc_body,
        mesh=mesh,
        compiler_params=pltpu.CompilerParams(use_tc_tiling_on_sc=False),
        out_type=jax.ShapeDtypeStruct((BATCH,), jnp.float32),
        scratch_types=[
            pltpu.VMEM((B_PER_W,), jnp.int32),
            pltpu.VMEM((B_PER_W,), jnp.int32),
            pltpu.VMEM((B_PER_W, DIM), jnp.float32),
            pltpu.VMEM((B_PER_W, DIM), jnp.float32),
            pltpu.VMEM((B_PER_W,), jnp.float32),
            pltpu.SemaphoreType.DMA,
            pltpu.SemaphoreType.DMA,
            pltpu.SemaphoreType.DMA,
            pltpu.SemaphoreType.DMA,
            pltpu.SemaphoreType.DMA,
        ],
    )
    return kern(context_idxs, target_idxs, cxt_table, emb_table)


def kernel(context_idxs, target_idxs, cxt_table, emb_table):
    out = _sc_call(context_idxs, target_idxs, cxt_table, emb_table)
    return out.reshape(-1, 1)


# speedup vs baseline: 1.0057x; 1.0057x over previous
import jax
import jax.numpy as jnp
from jax import lax
from jax.experimental import pallas as pl
from jax.experimental.pallas import tpu as pltpu
from jax.experimental.pallas import tpu_sc as plsc

VOCAB = 1000
DIM = 64
BATCH = 4096

NC = 2
NS = 16
NW = NC * NS
LANES = 16
B_PER_W = BATCH // NW
GROUPS = B_PER_W // LANES
CHUNKS = DIM // LANES
HALF = B_PER_W // 2

_GATHER_DNUMS = lax.GatherDimensionNumbers(
    offset_dims=(), collapsed_slice_dims=(0,), start_index_map=(0,))


def _lane_perm(x, idx):
    return lax.gather(
        x, idx[:, None], _GATHER_DNUMS, slice_sizes=(1,),
        mode=lax.GatherScatterMode.PROMISE_IN_BOUNDS)


def _sc_body(ctx_idx_hbm, tgt_idx_hbm, cxt_hbm, emb_hbm, out_hbm,
             cidx_v, tidx_v, crows_v, trows_v, out_v,
             sem_i, sem_a, sem_b):
    wid = lax.axis_index("s") * NC + lax.axis_index("c")
    base = wid * B_PER_W

    ci = pltpu.async_copy(ctx_idx_hbm.at[pl.ds(base, B_PER_W)], cidx_v, sem_i)
    ti = pltpu.async_copy(tgt_idx_hbm.at[pl.ds(base, B_PER_W)], tidx_v, sem_i)
    ci.wait()
    ti.wait()
    cp = []
    for h, sem in ((0, sem_a), (1, sem_b)):
        sl = pl.ds(h * HALF, HALF)
        cp.append(pltpu.async_copy(cxt_hbm.at[cidx_v.at[sl]],
                                   crows_v.at[sl], sem))
        cp.append(pltpu.async_copy(emb_hbm.at[tidx_v.at[sl]],
                                   trows_v.at[sl], sem))

    lane = lax.iota(jnp.int32, LANES)
    bitrev = (((lane & 1) << 3) | ((lane & 2) << 1)
              | ((lane & 4) >> 1) | ((lane & 8) >> 3))

    def combine(a, b, sh):
        keep = (lane & sh) == 0
        return jnp.where(keep, a + _lane_perm(a, lane ^ sh),
                         b + _lane_perm(b, lane ^ sh))

    def do_group(g):
        ps = []
        for j in range(LANES):
            r = g * LANES + j
            p = crows_v[r, pl.ds(0, LANES)] * trows_v[r, pl.ds(0, LANES)]
            for k in range(1, CHUNKS):
                p = p + (crows_v[r, pl.ds(k * LANES, LANES)]
                         * trows_v[r, pl.ds(k * LANES, LANES)])
            ps.append(p)
        for sh in (8, 4, 2, 1):
            ps = [combine(ps[2 * i], ps[2 * i + 1], sh)
                  for i in range(len(ps) // 2)]
        out_v[pl.ds(g * LANES, LANES)] = _lane_perm(ps[0], bitrev)

    def loop_body(g, carry):
        @pl.when(g == 0)
        def _():
            cp[0].wait()
            cp[1].wait()

        @pl.when(g == GROUPS // 2)
        def _():
            cp[2].wait()
            cp[3].wait()

        do_group(g)
        return carry

    lax.fori_loop(0, GROUPS, loop_body, 0)

    pltpu.sync_copy(out_v, out_hbm.at[pl.ds(base, B_PER_W)])


@jax.jit
def _sc_call(context_idxs, target_idxs, cxt_table, emb_table):
    mesh = plsc.VectorSubcoreMesh(core_axis_name="c", subcore_axis_name="s")
    kern = pl.kernel(
        _sc_body,
        mesh=mesh,
        compiler_params=pltpu.CompilerParams(use_tc_tiling_on_sc=False),
        out_type=jax.ShapeDtypeStruct((BATCH,), jnp.float32),
        scratch_types=[
            pltpu.VMEM((B_PER_W,), jnp.int32),
            pltpu.VMEM((B_PER_W,), jnp.int32),
            pltpu.VMEM((B_PER_W, DIM), jnp.float32),
            pltpu.VMEM((B_PER_W, DIM), jnp.float32),
            pltpu.VMEM((B_PER_W,), jnp.float32),
            pltpu.SemaphoreType.DMA,
            pltpu.SemaphoreType.DMA,
            pltpu.SemaphoreType.DMA,
        ],
    )
    return kern(context_idxs, target_idxs, cxt_table, emb_table)


def kernel(context_idxs, target_idxs, cxt_table, emb_table):
    out = _sc_call(context_idxs, target_idxs, cxt_table, emb_table)
    return out.reshape(-1, 1)
